# baseline (speedup 1.0000x reference)
import functools

import jax
import jax.numpy as jnp
from jax import lax
from jax.experimental import pallas as pl
from jax.experimental.pallas import tpu as pltpu
from jax.experimental.pallas import tpu_sc as plsc

N = 16384
C = 1000

_info = plsc.get_sparse_core_info()
_NC = _info.num_cores
_NS = _info.num_subcores
_L = _info.num_lanes
_NW = _NC * _NS
_ROWS_PER_W = N // _NW
_CHUNK = 32
_NCHUNK = _ROWS_PER_W // _CHUNK


def _one_hot_body(x_hbm, out_hbm, x_v, buf0, buf1, sem0, sem1):
    wid = lax.axis_index("s") * _NC + lax.axis_index("c")
    row0 = wid * _ROWS_PER_W

    pltpu.sync_copy(x_hbm.at[pl.ds(row0, _ROWS_PER_W)], x_v)

    zeros = jnp.zeros((_L,), jnp.int32)

    def _zero(i, _):
        buf0[pl.ds(i * _L, _L)] = zeros
        buf1[pl.ds(i * _L, _L)] = zeros
        return 0

    lax.fori_loop(0, _CHUNK * C // _L, _zero, 0)

    iota = lax.iota(jnp.int32, _L)
    ones = jnp.ones((_L,), jnp.int32)
    bufs = (buf0, buf1)
    sems = (sem0, sem1)
    copies = [None, None]

    for c in range(_NCHUNK):
        b = c % 2
        if c >= 2:
            copies[b].wait()

            for g in range(_CHUNK // _L):
                xv = x_v[pl.ds((c - 2) * _CHUNK + g * _L, _L)]
                pos = (g * _L + iota) * C + xv
                plsc.store_scatter(bufs[b], [pos], zeros)
        for g in range(_CHUNK // _L):
            xv = x_v[pl.ds(c * _CHUNK + g * _L, _L)]
            pos = (g * _L + iota) * C + xv
            plsc.store_scatter(bufs[b], [pos], ones)
        dst = out_hbm.at[pl.ds((row0 + c * _CHUNK) * C, _CHUNK * C)]
        copies[b] = pltpu.async_copy(bufs[b], dst, sems[b])

    copies[0].wait()
    copies[1].wait()


_one_hot = pl.kernel(
    _one_hot_body,
    out_type=jax.ShapeDtypeStruct((N * C,), jnp.int32),
    mesh=plsc.VectorSubcoreMesh(core_axis_name="c", subcore_axis_name="s"),
    scratch_types=[
        pltpu.VMEM((_ROWS_PER_W,), jnp.int32),
        pltpu.VMEM((_CHUNK * C,), jnp.int32),
        pltpu.VMEM((_CHUNK * C,), jnp.int32),
        pltpu.SemaphoreType.DMA,
        pltpu.SemaphoreType.DMA,
    ],
    compiler_params=pltpu.CompilerParams(needs_layout_passes=False),
)


@jax.jit
def kernel(x):
    return _one_hot(x).reshape(N, C)
